# baseline (speedup 1.0000x reference)
import jax
import jax.numpy as jnp
from jax import lax
from jax.experimental import pallas as pl
from jax.experimental.pallas import tpu as pltpu
from jax.experimental.pallas import tpu_sc as plsc

BATCH = 64
SEQ_LEN = 512
N_EMBD = 512

NC = 2
NS = 16
L = 16
NW = NC * NS
P_PER_W = SEQ_LEN // NW
G = 2
NSTEPS = BATCH // G
NBUF = 4
LEAD = 2
CCHUNKS = N_EMBD // L


def _embed_body(x_hbm, tok_hbm, pos_hbm, out_hbm,
                idx_v, pos_v, gbuf, gsem, osem):
    wid = lax.axis_index("s") * NC + lax.axis_index("c")
    p0 = wid * P_PER_W

    c0 = (wid // 8) * 128
    coff = (wid % 8) * P_PER_W
    pltpu.sync_copy(x_hbm.at[:, pl.ds(c0, 128)], idx_v)
    pltpu.sync_copy(pos_hbm.at[pl.ds(p0, P_PER_W), :], pos_v)

    def gathers(t, slot):
        return [
            pltpu.make_async_copy(
                tok_hbm.at[idx_v.at[t * G + j, pl.ds(coff, P_PER_W)]],
                gbuf.at[slot, j],
                gsem.at[slot])
            for j in range(G)
        ]

    def out_dma(t, slot):
        return pltpu.make_async_copy(
            gbuf.at[slot], out_hbm.at[pl.ds(t * G, G), pl.ds(p0, P_PER_W), :],
            osem.at[slot])

    for t in range(LEAD):
        for d in gathers(t, t):
            d.start()

    def group(g, _):
        for k in range(NBUF):
            t = g * NBUF + k
            for d in gathers(t, k):
                d.wait()

            def add_chunk(c, _):
                cs = pl.ds(c * L, L)
                for p in range(P_PER_W):
                    posv = pos_v[p, cs]
                    for j in range(G):
                        gbuf[k, j, p, cs] = gbuf[k, j, p, cs] + posv
                return ()
            lax.fori_loop(0, CCHUNKS, add_chunk, ())

            out_dma(t, k).start()

            kg = (k + LEAD) % NBUF

            @pl.when(t + LEAD < NSTEPS)
            def _():
                @pl.when(t >= LEAD)
                def _():
                    out_dma(t - LEAD, kg).wait()
                for d in gathers(t + LEAD, kg):
                    d.start()
        return ()

    lax.fori_loop(0, NSTEPS // NBUF, group, ())

    for t in range(NSTEPS - 2 * LEAD, NSTEPS):
        out_dma(t, t % NBUF).wait()


@jax.jit
def _embed(x, token_table, position_table):
    mesh = plsc.VectorSubcoreMesh(core_axis_name="c", subcore_axis_name="s")
    return pl.kernel(
        _embed_body,
        out_type=jax.ShapeDtypeStruct((BATCH, SEQ_LEN, N_EMBD), jnp.float32),
        mesh=mesh,
        scratch_types=[
            pltpu.VMEM((BATCH, 128), jnp.int32),
            pltpu.VMEM((P_PER_W, N_EMBD), jnp.float32),
            pltpu.VMEM((NBUF, G, P_PER_W, N_EMBD), jnp.float32),
            pltpu.SemaphoreType.DMA((NBUF,)),
            pltpu.SemaphoreType.DMA((NBUF,)),
        ],
    )(x, token_table, position_table)


def kernel(x, token_table, position_table):
    return _embed(x, token_table, position_table)
